# Initial kernel scaffold; baseline (speedup 1.0000x reference)
import jax
import jax.numpy as jnp
from jax.experimental import pallas as pl


def kernel(N, idx):
    raise NotImplementedError("write your pallas kernel here")


# speedup vs baseline: 1.3806x; 1.3806x over previous
import functools

import jax
import jax.numpy as jnp
from jax import lax
from jax.experimental import pallas as pl
from jax.experimental.pallas import tpu as pltpu
from jax.experimental.pallas import tpu_sc as plsc

VOCAB = 1000
BATCH = 4096
HIST = 20
PRIOR = 1.0


_DPAD = 1024


def _table_body(n_ref, p_ref):
    n = n_ref[...] + PRIOR
    s = jnp.sum(n, axis=1, keepdims=True)
    p = jnp.log(n / s)
    p = jnp.maximum(p, -1.0e6)
    p_ref[...] = jnp.pad(p, ((0, 0), (0, _DPAD - VOCAB)))


def _compute_table(N):
    return pl.pallas_call(
        _table_body,
        out_shape=jax.ShapeDtypeStruct((VOCAB, _DPAD), jnp.float32),
        in_specs=[pl.BlockSpec(memory_space=pltpu.VMEM)],
        out_specs=pl.BlockSpec(memory_space=pltpu.VMEM),
    )(N)


_B = BATCH * HIST


def _make_gather(V, D, B):
    info = plsc.get_sparse_core_info()
    NC, NS = info.num_cores, info.num_subcores
    NW = NC * NS
    assert B % NW == 0
    per_w = B // NW
    CH = 64
    assert per_w % CH == 0
    n_ch = per_w // CH
    mesh = plsc.VectorSubcoreMesh(core_axis_name="c", subcore_axis_name="s")

    @functools.partial(
        pl.kernel,
        mesh=mesh,
        out_type=jax.ShapeDtypeStruct((B, _DPAD), jnp.float32),
        scratch_types=[
            pltpu.VMEM((CH,), jnp.int32),
            pltpu.VMEM((CH, _DPAD), jnp.float32),
            pltpu.SemaphoreType.DMA,
        ],
    )
    def gather(table_hbm, idx_hbm, out_hbm, idx_v, rows_v, sem):
        wid = lax.axis_index("s") * NC + lax.axis_index("c")
        base = wid * per_w

        def body(i, carry):
            off = base + i * CH
            pltpu.sync_copy(idx_hbm.at[pl.ds(off, CH)], idx_v)
            pltpu.async_copy(table_hbm.at[idx_v], rows_v, sem).wait()
            pltpu.sync_copy(rows_v, out_hbm.at[pl.ds(off, CH)])
            return carry

        lax.fori_loop(0, n_ch, body, 0)

    return gather


_gather = _make_gather(VOCAB, VOCAB, _B)


def kernel(N, idx):
    p = _compute_table(N.astype(jnp.float32))
    flat = idx.reshape(-1).astype(jnp.int32)
    out = _gather(p, flat)
    return out[:, :VOCAB].reshape(BATCH, HIST, VOCAB)


# speedup vs baseline: 1.4158x; 1.0255x over previous
import functools

import jax
import jax.numpy as jnp
from jax import lax
from jax.experimental import pallas as pl
from jax.experimental.pallas import tpu as pltpu
from jax.experimental.pallas import tpu_sc as plsc

VOCAB = 1000
BATCH = 4096
HIST = 20
PRIOR = 1.0


_DPAD = 1024


def _table_body(n_ref, p_ref):
    n = n_ref[...] + PRIOR
    s = jnp.sum(n, axis=1, keepdims=True)
    p = jnp.log(n / s)
    p = jnp.maximum(p, -1.0e6)
    p_ref[...] = jnp.pad(p, ((0, 0), (0, _DPAD - VOCAB)))


def _compute_table(N):
    return pl.pallas_call(
        _table_body,
        out_shape=jax.ShapeDtypeStruct((VOCAB, _DPAD), jnp.float32),
        in_specs=[pl.BlockSpec(memory_space=pltpu.VMEM)],
        out_specs=pl.BlockSpec(memory_space=pltpu.VMEM),
    )(N)


_B = BATCH * HIST


def _make_gather(V, D, B):
    info = plsc.get_sparse_core_info()
    NC, NS = info.num_cores, info.num_subcores
    NW = NC * NS
    assert B % NW == 0
    per_w = B // NW
    CH = 40
    assert per_w % (2 * CH) == 0
    n2 = per_w // (2 * CH)
    mesh = plsc.VectorSubcoreMesh(core_axis_name="c", subcore_axis_name="s")

    @functools.partial(
        pl.kernel,
        mesh=mesh,
        out_type=jax.ShapeDtypeStruct((B, _DPAD), jnp.float32),
        scratch_types=[
            pltpu.VMEM((per_w,), jnp.int32),
            pltpu.VMEM((CH, _DPAD), jnp.float32),
            pltpu.VMEM((CH, _DPAD), jnp.float32),
            pltpu.SemaphoreType.DMA,
            pltpu.SemaphoreType.DMA,
            pltpu.SemaphoreType.DMA,
            pltpu.SemaphoreType.DMA,
        ],
    )
    def gather(table_hbm, idx_hbm, out_hbm, idx_v, buf0, buf1,
               sg0, sg1, so0, so1):
        wid = lax.axis_index("s") * NC + lax.axis_index("c")
        base = wid * per_w
        pltpu.sync_copy(idx_hbm.at[pl.ds(base, per_w)], idx_v)

        def wait_bytes(sem):
            pltpu.make_async_copy(out_hbm.at[pl.ds(0, CH)], buf0, sem).wait()

        pltpu.async_copy(table_hbm.at[idx_v.at[pl.ds(0, CH)]], buf0, sg0)

        def body(k, carry):
            c0 = 2 * k * CH
            c1 = c0 + CH

            @pl.when(k > 0)
            def _():
                wait_bytes(so1)
            pltpu.async_copy(
                table_hbm.at[idx_v.at[pl.ds(c1, CH)]], buf1, sg1)
            wait_bytes(sg0)
            pltpu.async_copy(buf0, out_hbm.at[pl.ds(base + c0, CH)], so0)
            wait_bytes(sg1)
            pltpu.async_copy(buf1, out_hbm.at[pl.ds(base + c1, CH)], so1)
            wait_bytes(so0)

            @pl.when(k + 1 < n2)
            def _():
                pltpu.async_copy(
                    table_hbm.at[idx_v.at[pl.ds(c1 + CH, CH)]], buf0, sg0)
            return carry

        lax.fori_loop(0, n2, body, 0)
        wait_bytes(so1)

    return gather


_gather = _make_gather(VOCAB, VOCAB, _B)


def kernel(N, idx):
    p = _compute_table(N.astype(jnp.float32))
    flat = idx.reshape(-1).astype(jnp.int32)
    out = _gather(p, flat)
    return out[:, :VOCAB].reshape(BATCH, HIST, VOCAB)


# speedup vs baseline: 1.4183x; 1.0017x over previous
import functools

import jax
import jax.numpy as jnp
from jax import lax
from jax.experimental import pallas as pl
from jax.experimental.pallas import tpu as pltpu
from jax.experimental.pallas import tpu_sc as plsc

VOCAB = 1000
BATCH = 4096
HIST = 20
PRIOR = 1.0


def _table_body(n_ref, p_ref):
    n = n_ref[...] + PRIOR
    s = jnp.sum(n, axis=1, keepdims=True)
    p = jnp.log(n / s)
    p_ref[...] = jnp.maximum(p, -1.0e6)


def _compute_table(N):
    return pl.pallas_call(
        _table_body,
        out_shape=jax.ShapeDtypeStruct((VOCAB, VOCAB), jnp.float32),
        in_specs=[pl.BlockSpec(memory_space=pltpu.VMEM)],
        out_specs=pl.BlockSpec(memory_space=pltpu.VMEM),
    )(N)


_B = BATCH * HIST


def _make_gather(V, D, B):
    info = plsc.get_sparse_core_info()
    NC, NS = info.num_cores, info.num_subcores
    NW = NC * NS
    assert B % NW == 0
    per_w = B // NW
    CH = 40
    assert per_w % (2 * CH) == 0
    n2 = per_w // (2 * CH)
    mesh = plsc.VectorSubcoreMesh(core_axis_name="c", subcore_axis_name="s")

    @functools.partial(
        pl.kernel,
        mesh=mesh,
        out_type=jax.ShapeDtypeStruct((B, D), jnp.float32),
        scratch_types=[
            pltpu.VMEM((per_w,), jnp.int32),
            pltpu.VMEM((CH, D), jnp.float32),
            pltpu.VMEM((CH, D), jnp.float32),
            pltpu.SemaphoreType.DMA,
            pltpu.SemaphoreType.DMA,
            pltpu.SemaphoreType.DMA,
            pltpu.SemaphoreType.DMA,
        ],
        compiler_params=pltpu.CompilerParams(use_tc_tiling_on_sc=False),
    )
    def gather(table_hbm, idx_hbm, out_hbm, idx_v, buf0, buf1,
               sg0, sg1, so0, so1):
        wid = lax.axis_index("s") * NC + lax.axis_index("c")
        base = wid * per_w
        pltpu.sync_copy(idx_hbm.at[pl.ds(base, per_w)], idx_v)

        def wait_bytes(sem):
            pltpu.make_async_copy(out_hbm.at[pl.ds(0, CH)], buf0, sem).wait()

        pltpu.async_copy(table_hbm.at[idx_v.at[pl.ds(0, CH)]], buf0, sg0)

        def body(k, carry):
            c0 = 2 * k * CH
            c1 = c0 + CH

            @pl.when(k > 0)
            def _():
                wait_bytes(so1)
            pltpu.async_copy(
                table_hbm.at[idx_v.at[pl.ds(c1, CH)]], buf1, sg1)
            wait_bytes(sg0)
            pltpu.async_copy(buf0, out_hbm.at[pl.ds(base + c0, CH)], so0)
            wait_bytes(sg1)
            pltpu.async_copy(buf1, out_hbm.at[pl.ds(base + c1, CH)], so1)
            wait_bytes(so0)

            @pl.when(k + 1 < n2)
            def _():
                pltpu.async_copy(
                    table_hbm.at[idx_v.at[pl.ds(c1 + CH, CH)]], buf0, sg0)
            return carry

        lax.fori_loop(0, n2, body, 0)
        wait_bytes(so1)

    return gather


_gather = _make_gather(VOCAB, VOCAB, _B)


def kernel(N, idx):
    p = _compute_table(N.astype(jnp.float32))
    flat = idx.reshape(-1).astype(jnp.int32)
    out = _gather(p, flat)
    return out.reshape(BATCH, HIST, VOCAB)
